# Initial kernel scaffold; baseline (speedup 1.0000x reference)
import functools

import jax
import jax.numpy as jnp
from jax import lax
from jax.experimental import pallas as pl
from jax.experimental.pallas import tpu as pltpu
from jax.experimental.pallas import tpu_sc as plsc

N = 10000
E = 320000
D = 128

NC = 2
NS = 16
NW = NC * NS
EDGES_PER_TILE = E // NW
K = 80
CHUNKS = EDGES_PER_TILE // K
ROWS_PER_TILE = N // NS
ZROWS = 125


def _matmul_body(x_ref, w_ref, o_ref):
    o_ref[...] = jnp.dot(x_ref[...], w_ref[...], preferred_element_type=jnp.float32)


def _supports(inputs, W):
    return pl.pallas_call(
        _matmul_body,
        grid=(10,),
        in_specs=[
            pl.BlockSpec((N // 10, D), lambda i: (i, 0)),
            pl.BlockSpec((D, D), lambda i: (0, 0)),
        ],
        out_specs=pl.BlockSpec((N // 10, D), lambda i: (i, 0)),
        out_shape=jax.ShapeDtypeStruct((N, D), jnp.float32),
    )(inputs, W)


def _spmm_body(sup_hbm, src_hbm, dst_hbm, adj_hbm, out_hbm,
               src_v, dst_v, adj_v, rows_v, stage_v, accum, sem):
    c = lax.axis_index("c")
    s = lax.axis_index("s")
    wid = c * NS + s

    zeros16 = jnp.zeros((16,), jnp.float32)

    def _zero_row(i, _):
        for j in range(D // 16):
            stage_v[i, pl.ds(j * 16, 16)] = zeros16
        return 0

    lax.fori_loop(0, ZROWS, _zero_row, 0)
    for t in range(ROWS_PER_TILE // ZROWS):
        pltpu.sync_copy(stage_v, accum.at[pl.ds(s * ROWS_PER_TILE + t * ZROWS, ZROWS)])
    plsc.subcore_barrier()

    base = wid * EDGES_PER_TILE

    def _chunk(k, _):
        off = base + k * K
        pltpu.sync_copy(src_hbm.at[pl.ds(off, K)], src_v)
        pltpu.sync_copy(dst_hbm.at[pl.ds(off, K)], dst_v)
        pltpu.sync_copy(adj_hbm.at[pl.ds(off, K)], adj_v)
        pltpu.async_copy(sup_hbm.at[src_v], rows_v, sem).wait()

        def _scale(e, _):
            a = adj_v[e]
            for j in range(D // 16):
                sl = pl.ds(j * 16, 16)
                rows_v[e, sl] = rows_v[e, sl] * a
            return 0

        lax.fori_loop(0, K, _scale, 0)
        pltpu.sync_copy(rows_v, accum.at[dst_v], add=True)
        return 0

    lax.fori_loop(0, CHUNKS, _chunk, 0)
    plsc.subcore_barrier()

    for t in range(ROWS_PER_TILE // ZROWS):
        r = s * ROWS_PER_TILE + t * ZROWS
        pltpu.sync_copy(accum.at[pl.ds(r, ZROWS)], stage_v)
        pltpu.sync_copy(stage_v, out_hbm.at[c, pl.ds(r, ZROWS)])


_spmm = functools.partial(
    pl.kernel,
    out_type=jax.ShapeDtypeStruct((NC, N, D), jnp.float32),
    mesh=plsc.VectorSubcoreMesh(
        core_axis_name="c", subcore_axis_name="s", num_cores=NC, num_subcores=NS),
    scratch_types=[
        pltpu.VMEM((K,), jnp.int32),
        pltpu.VMEM((K,), jnp.int32),
        pltpu.VMEM((K,), jnp.float32),
        pltpu.VMEM((K, D), jnp.float32),
        pltpu.VMEM((ZROWS, D), jnp.float32),
        pltpu.VMEM_SHARED((N, D), jnp.float32),
        pltpu.SemaphoreType.DMA,
    ],
)(_spmm_body)


def _combine_body(p_ref, b_ref, o_ref):
    o_ref[...] = p_ref[0] + p_ref[1] + b_ref[...]


def _combine(partials, bias):
    return pl.pallas_call(
        _combine_body,
        grid=(10,),
        in_specs=[
            pl.BlockSpec((NC, N // 10, D), lambda i: (0, i, 0)),
            pl.BlockSpec((1, D), lambda i: (0, 0)),
        ],
        out_specs=pl.BlockSpec((N // 10, D), lambda i: (i, 0)),
        out_shape=jax.ShapeDtypeStruct((N, D), jnp.float32),
    )(partials, bias.reshape(1, D))


@jax.jit
def kernel(inputs, edge_index, adj_values, W, bias):
    supports = _supports(inputs, W)
    src = edge_index[0]
    dst = edge_index[1]
    partials = _spmm(supports, src, dst, adj_values)
    return _combine(partials, bias)


# speedup vs baseline: 4.4578x; 4.4578x over previous
import functools

import jax
import jax.numpy as jnp
from jax import lax
from jax.experimental import pallas as pl
from jax.experimental.pallas import tpu as pltpu
from jax.experimental.pallas import tpu_sc as plsc

N = 10000
NP = 10240
E = 320000
D = 128

NC = 2
NS = 16
NW = NC * NS
EDGES_PER_TILE = E // NW
K = 80
CHUNKS = EDGES_PER_TILE // K
ROWS_PER_TILE = NP // NS
ZROWS = 128


def _matmul_body(x_ref, w_ref, o_ref):
    o_ref[...] = jnp.dot(x_ref[...], w_ref[...], preferred_element_type=jnp.float32)


def _supports(inputs, W):
    return pl.pallas_call(
        _matmul_body,
        grid=(10,),
        in_specs=[
            pl.BlockSpec((N // 10, D), lambda i: (i, 0)),
            pl.BlockSpec((D, D), lambda i: (0, 0)),
        ],
        out_specs=pl.BlockSpec((N // 10, D), lambda i: (i, 0)),
        out_shape=jax.ShapeDtypeStruct((N, D), jnp.float32),
    )(inputs, W)


def _spmm_body(sup_hbm, src_hbm, dst_hbm, adj_hbm, out_hbm,
               src_v, dst_v, adj_v, rows_v, stage_v, accum, sem):
    c = lax.axis_index("c")
    s = lax.axis_index("s")
    wid = c * NS + s

    zeros16 = jnp.zeros((16,), jnp.float32)

    def _zero_row(i, _):
        for j in range(D // 16):
            stage_v[i, pl.ds(j * 16, 16)] = zeros16
        return 0

    lax.fori_loop(0, ZROWS, _zero_row, 0)
    for t in range(ROWS_PER_TILE // ZROWS):
        pltpu.sync_copy(stage_v, accum.at[pl.ds(s * ROWS_PER_TILE + t * ZROWS, ZROWS)])
    plsc.subcore_barrier()

    base = wid * EDGES_PER_TILE

    def _chunk(k, _):
        off = base + k * K
        pltpu.sync_copy(src_hbm.at[pl.ds(off, K)], src_v)
        pltpu.sync_copy(dst_hbm.at[pl.ds(off, K)], dst_v)
        pltpu.sync_copy(adj_hbm.at[pl.ds(off, K)], adj_v)
        pltpu.async_copy(sup_hbm.at[src_v], rows_v, sem).wait()

        def _scale(g, _):
            av = adj_v[pl.ds(g * 16, 16)]
            for l in range(16):
                a = av[l]
                e = g * 16 + l
                for j in range(D // 16):
                    sl = pl.ds(j * 16, 16)
                    rows_v[e, sl] = rows_v[e, sl] * a
            return 0

        lax.fori_loop(0, K // 16, _scale, 0)
        pltpu.sync_copy(rows_v, accum.at[dst_v], add=True)
        return 0

    lax.fori_loop(0, CHUNKS, _chunk, 0)
    plsc.subcore_barrier()

    for t in range(ROWS_PER_TILE // ZROWS):
        r = s * ROWS_PER_TILE + t * ZROWS
        pltpu.sync_copy(accum.at[pl.ds(r, ZROWS)], stage_v)
        pltpu.sync_copy(stage_v, out_hbm.at[c, pl.ds(r, ZROWS)])


_spmm = functools.partial(
    pl.kernel,
    out_type=jax.ShapeDtypeStruct((NC, NP, D), jnp.float32),
    mesh=plsc.VectorSubcoreMesh(
        core_axis_name="c", subcore_axis_name="s", num_cores=NC, num_subcores=NS),
    scratch_types=[
        pltpu.VMEM((K,), jnp.int32),
        pltpu.VMEM((K,), jnp.int32),
        pltpu.VMEM((K,), jnp.float32),
        pltpu.VMEM((K, D), jnp.float32),
        pltpu.VMEM((ZROWS, D), jnp.float32),
        pltpu.VMEM_SHARED((NP, D), jnp.float32),
        pltpu.SemaphoreType.DMA,
    ],
)(_spmm_body)


def _combine_body(p_ref, b_ref, o_ref):
    o_ref[...] = p_ref[0] + p_ref[1] + b_ref[...]


def _combine(partials, bias):
    return pl.pallas_call(
        _combine_body,
        grid=(10,),
        in_specs=[
            pl.BlockSpec((NC, N // 10, D), lambda i: (0, i, 0)),
            pl.BlockSpec((1, D), lambda i: (0, 0)),
        ],
        out_specs=pl.BlockSpec((N // 10, D), lambda i: (i, 0)),
        out_shape=jax.ShapeDtypeStruct((N, D), jnp.float32),
    )(partials, bias.reshape(1, D))


@jax.jit
def kernel(inputs, edge_index, adj_values, W, bias):
    supports = _supports(inputs, W)
    src = edge_index[0]
    dst = edge_index[1]
    partials = _spmm(supports, src, dst, adj_values)
    return _combine(partials, bias)
